# Initial kernel scaffold; baseline (speedup 1.0000x reference)
import jax
import jax.numpy as jnp
from jax.experimental import pallas as pl


def kernel(x):
    raise NotImplementedError("write your pallas kernel here")


# speedup vs baseline: 12.0152x; 12.0152x over previous
import functools

import jax
import jax.numpy as jnp
from jax import lax
from jax.experimental import pallas as pl
from jax.experimental.pallas import tpu as pltpu
from jax.experimental.pallas import tpu_sc as plsc

D = 16
N = 1_000_000
NB = 1024
EPS = float(jnp.finfo(jnp.float32).eps)
TOT = float(D * N)

NC = 2
NS = 16
NW = NC * NS
PER_W = (D * N) // NW
C = 50_000
NCH = PER_W // C
VPC = C // 16
UNROLL = 5
INNER = VPC // UNROLL

_mesh = plsc.VectorSubcoreMesh(core_axis_name="c", subcore_axis_name="s")


def _wid():
    return lax.axis_index("s") * NC + lax.axis_index("c")


def _bcast_last_max(v):
    return plsc.cummax(lax.rev(plsc.cummax(v), (0,)))


def _reduce_minmax(mm_v):
    mn = mm_v[pl.ds(0, 16)]
    mx = mm_v[pl.ds(16, 16)]
    for j in range(1, NW):
        mn = jnp.minimum(mn, mm_v[pl.ds(32 * j, 16)])
        mx = jnp.maximum(mx, mm_v[pl.ds(32 * j + 16, 16)])
    lo_v = -_bcast_last_max(-mn)
    hi_v = _bcast_last_max(mx)
    return lo_v, hi_v


@functools.partial(
    pl.kernel,
    mesh=_mesh,
    compiler_params=pltpu.CompilerParams(needs_layout_passes=False),
    out_type=jax.ShapeDtypeStruct((NW * 32,), jnp.float32),
    scratch_types=[
        pltpu.VMEM((C,), jnp.float32),
        pltpu.VMEM((C,), jnp.float32),
        pltpu.VMEM((32,), jnp.float32),
        pltpu.SemaphoreType.DMA,
        pltpu.SemaphoreType.DMA,
    ],
)
def _minmax_k(x_hbm, mm_hbm, buf0, buf1, stage, sem0, sem1):
    w = _wid()
    base = w * PER_W
    pltpu.async_copy(x_hbm.at[pl.ds(base, C)], buf0, sem0)

    def chunk_pair(g, carry):
        mn, mx = carry
        for b in range(2):
            c = 2 * g + b
            buf = buf0 if b == 0 else buf1
            sem = sem0 if b == 0 else sem1
            nbuf = buf1 if b == 0 else buf0
            nsem = sem1 if b == 0 else sem0

            @pl.when(c + 1 < NCH)
            def _():
                pltpu.async_copy(
                    x_hbm.at[pl.ds(base + (c + 1) * C, C)], nbuf, nsem)

            pltpu.make_async_copy(x_hbm.at[pl.ds(0, C)], buf, sem).wait()

            def body(i, carry2):
                mn2, mx2 = carry2
                for u in range(UNROLL):
                    v = buf[pl.ds((i * UNROLL + u) * 16, 16)]
                    mn2 = jnp.minimum(mn2, v)
                    mx2 = jnp.maximum(mx2, v)
                return mn2, mx2

            mn, mx = lax.fori_loop(0, INNER, body, (mn, mx))
        return mn, mx

    init = (jnp.full((16,), jnp.inf, jnp.float32),
            jnp.full((16,), -jnp.inf, jnp.float32))
    mn, mx = lax.fori_loop(0, NCH // 2, chunk_pair, init)
    stage[pl.ds(0, 16)] = mn
    stage[pl.ds(16, 16)] = mx
    pltpu.sync_copy(stage, mm_hbm.at[pl.ds(32 * w, 32)])


@functools.partial(
    pl.kernel,
    mesh=_mesh,
    compiler_params=pltpu.CompilerParams(needs_layout_passes=False),
    out_type=jax.ShapeDtypeStruct((NW * NB,), jnp.float32),
    scratch_types=[
        pltpu.VMEM((C,), jnp.float32),
        pltpu.VMEM((C,), jnp.float32),
        pltpu.VMEM((NB,), jnp.float32),
        pltpu.VMEM((NW * 32,), jnp.float32),
        pltpu.SemaphoreType.DMA,
        pltpu.SemaphoreType.DMA,
    ],
)
def _hist_k(x_hbm, mm_hbm, hist_hbm, buf0, buf1, hist_v, mm_v, sem0, sem1):
    w = _wid()
    base = w * PER_W
    pltpu.async_copy(x_hbm.at[pl.ds(base, C)], buf0, sem0)
    pltpu.sync_copy(mm_hbm, mm_v)
    lo_v, hi_v = _reduce_minmax(mm_v)
    width_v = (hi_v - lo_v) * (1.0 / NB)
    wpe_v = width_v + EPS

    zero = jnp.zeros((16,), jnp.float32)
    for i in range(NB // 16):
        hist_v[pl.ds(16 * i, 16)] = zero
    ones = jnp.full((16,), 1.0, jnp.float32)
    top = jnp.full((16,), NB - 1, jnp.int32)

    def chunk_pair(g, carry):
        for b in range(2):
            c = 2 * g + b
            buf = buf0 if b == 0 else buf1
            sem = sem0 if b == 0 else sem1
            nbuf = buf1 if b == 0 else buf0
            nsem = sem1 if b == 0 else sem0

            @pl.when(c + 1 < NCH)
            def _():
                pltpu.async_copy(
                    x_hbm.at[pl.ds(base + (c + 1) * C, C)], nbuf, nsem)

            pltpu.make_async_copy(x_hbm.at[pl.ds(0, C)], buf, sem).wait()

            def body(i, carry2):
                for u in range(UNROLL):
                    v = buf[pl.ds((i * UNROLL + u) * 16, 16)]
                    q = (v - lo_v) / wpe_v
                    idx = jnp.minimum(q.astype(jnp.int32), top)
                    plsc.addupdate_scatter(hist_v, [idx], ones)
                return carry2

            lax.fori_loop(0, INNER, body, 0)
        return carry

    lax.fori_loop(0, NCH // 2, chunk_pair, 0)
    pltpu.sync_copy(hist_v, hist_hbm.at[pl.ds(NB * w, NB)])


@functools.partial(
    pl.kernel,
    mesh=_mesh,
    compiler_params=pltpu.CompilerParams(needs_layout_passes=False),
    out_type=jax.ShapeDtypeStruct((1040,), jnp.float32),
    scratch_types=[
        pltpu.VMEM((NW * NB,), jnp.float32),
        pltpu.VMEM((NB,), jnp.float32),
        pltpu.VMEM((NB,), jnp.float32),
        pltpu.VMEM((1040,), jnp.float32),
        pltpu.VMEM((NW * 32,), jnp.float32),
    ],
)
def _final_k(hist_hbm, mm_hbm, q_hbm, hall, acc, cdfm, qout, mm_v):
    w = _wid()

    @pl.when(w == 0)
    def _():
        pltpu.sync_copy(hist_hbm, hall)
        pltpu.sync_copy(mm_hbm, mm_v)
        lo_v, hi_v = _reduce_minmax(mm_v)
        width_v = (hi_v - lo_v) * (1.0 / NB)

        def merge(i, carry):
            s = jnp.zeros((16,), jnp.float32)
            for j in range(NW):
                s = s + hall[pl.ds(j * NB + i * 16, 16)]
            acc[pl.ds(16 * i, 16)] = s
            return carry

        lax.fori_loop(0, NB // 16, merge, 0)

        def csum(i, carry):
            v = acc[pl.ds(16 * i, 16)]
            cs = plsc.cumsum(v) + carry
            cdfm[pl.ds(16 * i, 16)] = cs
            return _bcast_last_max(cs)

        lax.fori_loop(0, NB // 16, csum, jnp.zeros((16,), jnp.float32))

        lane = lax.iota(jnp.int32, 16)
        zero_i = jnp.zeros((16,), jnp.int32)
        zero_f = jnp.zeros((16,), jnp.float32)

        def interp(i, carry):
            k = lane + 16 * i
            tq = k.astype(jnp.float32) * (TOT / NB)
            lo_i = zero_i
            hi_i = jnp.full((16,), NB + 1, jnp.int32)
            for _ in range(11):
                mid = (lo_i + hi_i) >> 1
                cm = plsc.load_gather(cdfm, [jnp.maximum(mid - 1, zero_i)])
                cval = jnp.where(mid == 0, zero_f, cm)
                pred = cval < tq
                lo_i = jnp.where(pred, mid + 1, lo_i)
                hi_i = jnp.where(pred, hi_i, mid)
            ind = jnp.clip(lo_i - 1, 0, NB - 1)
            indf = ind.astype(jnp.float32)
            e1 = lo_v + indf * width_v
            e2 = lo_v + (indf + 1.0) * width_v
            cs_lo = plsc.load_gather(cdfm, [jnp.maximum(ind - 1, zero_i)])
            cs_lo = jnp.where(ind == 0, zero_f, cs_lo)
            cs_hi = plsc.load_gather(cdfm, [ind])
            t = k.astype(jnp.float32) * (1.0 / NB)
            slope = (e2 - e1) / (EPS + (cs_hi * (1.0 / TOT) - cs_lo * (1.0 / TOT)))
            qv = e1 + slope * (t - cs_lo * (1.0 / TOT))
            qout[pl.ds(16 * i, 16)] = qv
            return carry

        lax.fori_loop(0, 1040 // 16, interp, 0)
        pltpu.sync_copy(qout, q_hbm)


def kernel(x):
    xf = x.reshape(-1)
    mm = _minmax_k(xf)
    hists = _hist_k(xf, mm)
    q = _final_k(hists, mm)
    return q[:1025]


# speedup vs baseline: 134.6451x; 11.2063x over previous
import functools

import jax
import jax.numpy as jnp
from jax import lax
from jax.experimental import pallas as pl
from jax.experimental.pallas import tpu as pltpu
from jax.experimental.pallas import tpu_sc as plsc

D = 16
N = 1_000_000
NB = 1024
EPS = float(jnp.finfo(jnp.float32).eps)
TOT = float(D * N)

NC = 2
NS = 16
NW = NC * NS

CW = 62_464
CC = 7_808
NCH = CW // CC
VPR = CC // 16
UNROLL = 4
INNER = VPR // UNROLL
TAIL0 = 16 * CW
TAILW = N - TAIL0
TAIL_N = D * TAILW
TA = 7808
TB = TAIL_N - TA

_mesh = plsc.VectorSubcoreMesh(core_axis_name="c", subcore_axis_name="s")


def _wid():
    return lax.axis_index("s") * NC + lax.axis_index("c")


def _bcast_last_max(v):
    return plsc.cummax(lax.rev(plsc.cummax(v), (0,)))


def _reduce_minmax(mm_v):
    mn = mm_v[pl.ds(0, 16)]
    mx = mm_v[pl.ds(16, 16)]
    for j in range(1, NW):
        mn = jnp.minimum(mn, mm_v[pl.ds(32 * j, 16)])
        mx = jnp.maximum(mx, mm_v[pl.ds(32 * j + 16, 16)])
    lo_v = -_bcast_last_max(-mn)
    hi_v = _bcast_last_max(mx)
    return lo_v, hi_v


def _worker_slices(w):
    band = w & 1
    c16 = w >> 1
    r0 = pl.multiple_of(band * 8, 8)
    base_c = pl.multiple_of(c16 * CW, 128)
    return c16, r0, base_c


@functools.partial(
    pl.kernel,
    mesh=_mesh,
    compiler_params=pltpu.CompilerParams(needs_layout_passes=False),
    out_type=jax.ShapeDtypeStruct((NW * 32,), jnp.float32),
    scratch_types=[
        pltpu.VMEM((8, CC), jnp.float32),
        pltpu.VMEM((8, CC), jnp.float32),
        pltpu.VMEM((32,), jnp.float32),
        pltpu.SemaphoreType.DMA,
        pltpu.SemaphoreType.DMA,
    ],
)
def _minmax_k(x_hbm, xt_hbm, mm_hbm, buf0, buf1, stage, sem0, sem1):
    w = _wid()
    c16, r0, base_c = _worker_slices(w)
    pltpu.async_copy(
        x_hbm.at[pl.ds(r0, 8), pl.ds(base_c, CC)], buf0, sem0)

    def chunk_pair(g, carry):
        mn, mx = carry
        for b in range(2):
            c = 2 * g + b
            buf = buf0 if b == 0 else buf1
            sem = sem0 if b == 0 else sem1
            nbuf = buf1 if b == 0 else buf0
            nsem = sem1 if b == 0 else sem0

            @pl.when(c + 1 < NCH)
            def _():
                col = pl.multiple_of(base_c + (c + 1) * CC, 128)
                pltpu.async_copy(
                    x_hbm.at[pl.ds(r0, 8), pl.ds(col, CC)], nbuf, nsem)

            pltpu.make_async_copy(
                x_hbm.at[pl.ds(0, 8), pl.ds(0, CC)], buf, sem).wait()

            for r in range(8):
                def body(i, carry2, _r=r, _buf=buf):
                    mn2, mx2 = carry2
                    for u in range(UNROLL):
                        v = _buf[_r, pl.ds((i * UNROLL + u) * 16, 16)]
                        mn2 = jnp.minimum(mn2, v)
                        mx2 = jnp.maximum(mx2, v)
                    return mn2, mx2

                mn, mx = lax.fori_loop(0, INNER, body, (mn, mx))
        return mn, mx

    init = (jnp.full((16,), jnp.inf, jnp.float32),
            jnp.full((16,), -jnp.inf, jnp.float32))
    mn, mx = lax.fori_loop(0, NCH // 2, chunk_pair, init)

    @pl.when(w == 0)
    def _():
        pltpu.sync_copy(xt_hbm.at[pl.ds(0, TA)], buf0.at[0, pl.ds(0, TA)])
        pltpu.sync_copy(xt_hbm.at[pl.ds(TA, TB)], buf0.at[1, pl.ds(0, TB)])

    mn_t, mx_t = (mn, mx)

    def tail_scan(_):
        def body(i, carry2):
            a, bx = carry2
            for u in range(UNROLL):
                v = buf0[0, pl.ds((i * UNROLL + u) * 16, 16)]
                a = jnp.minimum(a, v)
                bx = jnp.maximum(bx, v)
            return a, bx

        mn3, mx3 = lax.fori_loop(0, TA // 64, body, (mn_t, mx_t))

        def body_b(i, carry2):
            a, bx = carry2
            for u in range(UNROLL):
                v = buf0[1, pl.ds((i * UNROLL + u) * 16, 16)]
                a = jnp.minimum(a, v)
                bx = jnp.maximum(bx, v)
            return a, bx

        return lax.fori_loop(0, TB // 64, body_b, (mn3, mx3))

    mn, mx = lax.cond(w == 0, tail_scan, lambda _: (mn_t, mx_t), 0)

    stage[pl.ds(0, 16)] = mn
    stage[pl.ds(16, 16)] = mx
    pltpu.sync_copy(stage, mm_hbm.at[pl.ds(32 * w, 32)])


@functools.partial(
    pl.kernel,
    mesh=_mesh,
    compiler_params=pltpu.CompilerParams(needs_layout_passes=False),
    out_type=jax.ShapeDtypeStruct((NW * NB,), jnp.float32),
    scratch_types=[
        pltpu.VMEM((8, CC), jnp.float32),
        pltpu.VMEM((8, CC), jnp.float32),
        pltpu.VMEM((NB,), jnp.float32),
        pltpu.VMEM((NB,), jnp.float32),
        pltpu.VMEM((NB,), jnp.float32),
        pltpu.VMEM((NB,), jnp.float32),
        pltpu.VMEM((NW * 32,), jnp.float32),
        pltpu.SemaphoreType.DMA,
        pltpu.SemaphoreType.DMA,
    ],
)
def _hist_k(x_hbm, xt_hbm, mm_hbm, hist_hbm, buf0, buf1, h0, h1, h2, h3,
            mm_v, sem0, sem1):
    w = _wid()
    c16, r0, base_c = _worker_slices(w)
    pltpu.async_copy(
        x_hbm.at[pl.ds(r0, 8), pl.ds(base_c, CC)], buf0, sem0)
    pltpu.sync_copy(mm_hbm, mm_v)
    lo_v, hi_v = _reduce_minmax(mm_v)
    width_v = (hi_v - lo_v) * (1.0 / NB)
    wpe_v = width_v + EPS

    hists = (h0, h1, h2, h3)
    zero = jnp.zeros((16,), jnp.float32)
    for i in range(NB // 16):
        for h in hists:
            h[pl.ds(16 * i, 16)] = zero
    ones = jnp.full((16,), 1.0, jnp.float32)
    topf = jnp.full((16,), float(NB - 1), jnp.float32)

    def chunk_pair(g, carry):
        for b in range(2):
            c = 2 * g + b
            buf = buf0 if b == 0 else buf1
            sem = sem0 if b == 0 else sem1
            nbuf = buf1 if b == 0 else buf0
            nsem = sem1 if b == 0 else sem0

            @pl.when(c + 1 < NCH)
            def _():
                col = pl.multiple_of(base_c + (c + 1) * CC, 128)
                pltpu.async_copy(
                    x_hbm.at[pl.ds(r0, 8), pl.ds(col, CC)], nbuf, nsem)

            pltpu.make_async_copy(
                x_hbm.at[pl.ds(0, 8), pl.ds(0, CC)], buf, sem).wait()

            for r in range(8):
                @plsc.parallel_loop(0, INNER, 1, unroll=2)
                def body(i, _r=r, _buf=buf):
                    for u in range(UNROLL):
                        v = _buf[_r, pl.ds((i * UNROLL + u) * 16, 16)]
                        q = (v - lo_v) / wpe_v
                        idx = jnp.minimum(q, topf).astype(jnp.int32)
                        plsc.addupdate_scatter(hists[u], [idx], ones)
        return carry

    lax.fori_loop(0, NCH // 2, chunk_pair, 0)

    @pl.when(w == 0)
    def _():
        pltpu.sync_copy(xt_hbm.at[pl.ds(0, TA)], buf0.at[0, pl.ds(0, TA)])
        pltpu.sync_copy(xt_hbm.at[pl.ds(TA, TB)], buf0.at[1, pl.ds(0, TB)])
        for r, nvr in ((0, TA // 64), (1, TB // 64)):
            @plsc.parallel_loop(0, nvr, 1)
            def body(i, _r=r):
                for u in range(UNROLL):
                    v = buf0[_r, pl.ds((i * UNROLL + u) * 16, 16)]
                    q = (v - lo_v) / wpe_v
                    idx = jnp.minimum(q, topf).astype(jnp.int32)
                    plsc.addupdate_scatter(hists[u], [idx], ones)

    def merge(i, carry):
        s = (h0[pl.ds(16 * i, 16)] + h1[pl.ds(16 * i, 16)]
             + h2[pl.ds(16 * i, 16)] + h3[pl.ds(16 * i, 16)])
        h0[pl.ds(16 * i, 16)] = s
        return carry

    lax.fori_loop(0, NB // 16, merge, 0)
    pltpu.sync_copy(h0, hist_hbm.at[pl.ds(NB * w, NB)])


@functools.partial(
    pl.kernel,
    mesh=_mesh,
    compiler_params=pltpu.CompilerParams(needs_layout_passes=False),
    out_type=jax.ShapeDtypeStruct((1040,), jnp.float32),
    scratch_types=[
        pltpu.VMEM((NW * NB,), jnp.float32),
        pltpu.VMEM((NB,), jnp.float32),
        pltpu.VMEM((NB,), jnp.float32),
        pltpu.VMEM((1040,), jnp.float32),
        pltpu.VMEM((NW * 32,), jnp.float32),
    ],
)
def _final_k(hist_hbm, mm_hbm, q_hbm, hall, acc, cdfm, qout, mm_v):
    w = _wid()

    @pl.when(w == 0)
    def _():
        pltpu.sync_copy(hist_hbm, hall)
        pltpu.sync_copy(mm_hbm, mm_v)
        lo_v, hi_v = _reduce_minmax(mm_v)
        width_v = (hi_v - lo_v) * (1.0 / NB)

        def merge(i, carry):
            s = jnp.zeros((16,), jnp.float32)
            for j in range(NW):
                s = s + hall[pl.ds(j * NB + i * 16, 16)]
            acc[pl.ds(16 * i, 16)] = s
            return carry

        lax.fori_loop(0, NB // 16, merge, 0)

        def csum(i, carry):
            v = acc[pl.ds(16 * i, 16)]
            cs = plsc.cumsum(v) + carry
            cdfm[pl.ds(16 * i, 16)] = cs
            return _bcast_last_max(cs)

        lax.fori_loop(0, NB // 16, csum, jnp.zeros((16,), jnp.float32))

        lane = lax.iota(jnp.int32, 16)
        zero_i = jnp.zeros((16,), jnp.int32)
        zero_f = jnp.zeros((16,), jnp.float32)

        def interp(i, carry):
            k = lane + 16 * i
            tq = k.astype(jnp.float32) * (TOT / NB)
            lo_i = zero_i
            hi_i = jnp.full((16,), NB + 1, jnp.int32)
            for _ in range(11):
                mid = (lo_i + hi_i) >> 1
                cm = plsc.load_gather(cdfm, [jnp.maximum(mid - 1, zero_i)])
                cval = jnp.where(mid == 0, zero_f, cm)
                pred = cval < tq
                lo_i = jnp.where(pred, mid + 1, lo_i)
                hi_i = jnp.where(pred, hi_i, mid)
            ind = jnp.clip(lo_i - 1, 0, NB - 1)
            indf = ind.astype(jnp.float32)
            e1 = lo_v + indf * width_v
            e2 = lo_v + (indf + 1.0) * width_v
            cs_lo = plsc.load_gather(cdfm, [jnp.maximum(ind - 1, zero_i)])
            cs_lo = jnp.where(ind == 0, zero_f, cs_lo)
            cs_hi = plsc.load_gather(cdfm, [ind])
            t = k.astype(jnp.float32) * (1.0 / NB)
            slope = (e2 - e1) / (EPS + (cs_hi * (1.0 / TOT) - cs_lo * (1.0 / TOT)))
            qv = e1 + slope * (t - cs_lo * (1.0 / TOT))
            qout[pl.ds(16 * i, 16)] = qv
            return carry

        lax.fori_loop(0, 1040 // 16, interp, 0)
        pltpu.sync_copy(qout, q_hbm)


def kernel(x):
    xt = x[:, TAIL0:].reshape(-1)
    mm = _minmax_k(x, xt)
    hists = _hist_k(x, xt, mm)
    q = _final_k(hists, mm)
    return q[:1025]


# speedup vs baseline: 162.6029x; 1.2076x over previous
import functools

import jax
import jax.numpy as jnp
from jax import lax
from jax.experimental import pallas as pl
from jax.experimental.pallas import tpu as pltpu
from jax.experimental.pallas import tpu_sc as plsc

D = 16
N = 1_000_000
NB = 1024
EPS = float(jnp.finfo(jnp.float32).eps)
TOT = float(D * N)

NC = 2
NS = 16
NW = NC * NS

CW = 62_464
CC = 7_808
NCH = CW // CC
VPR = CC // 16
UNROLL = 4
INNER = VPR // UNROLL
TAIL0 = 16 * CW
TAILW = N - TAIL0
TAIL_N = D * TAILW
TA = 7808
TB = TAIL_N - TA

_mesh = plsc.VectorSubcoreMesh(core_axis_name="c", subcore_axis_name="s")


def _wid():
    return lax.axis_index("s") * NC + lax.axis_index("c")


def _bcast_last_max(v):
    return plsc.cummax(lax.rev(plsc.cummax(v), (0,)))


def _reduce_minmax(mm_v):
    return mm_v[pl.ds(0, 16)], mm_v[pl.ds(512, 16)]


def _worker_slices(w):
    band = w & 1
    c16 = w >> 1
    r0 = pl.multiple_of(band * 8, 8)
    base_c = pl.multiple_of(c16 * CW, 128)
    return c16, r0, base_c


_TCB = 62_592


@functools.partial(
    pl.pallas_call,
    grid=(16,),
    in_specs=[pl.BlockSpec((D, _TCB), lambda g: (0, g))],
    out_specs=pl.BlockSpec((1024,), lambda g: (0,)),
    out_shape=jax.ShapeDtypeStruct((1024,), jnp.float32),
)
def _minmax_tc(x_ref, mm_ref):
    g = pl.program_id(0)

    @pl.when(g == 0)
    def _():
        mm_ref[pl.ds(0, 512)] = jnp.full((512,), jnp.inf, jnp.float32)
        mm_ref[pl.ds(512, 512)] = jnp.full((512,), -jnp.inf, jnp.float32)

    v = x_ref[...]

    @pl.when(g < 15)
    def _():
        mm_ref[pl.ds(0, 512)] = jnp.minimum(mm_ref[pl.ds(0, 512)], jnp.min(v))
        mm_ref[pl.ds(512, 512)] = jnp.maximum(mm_ref[pl.ds(512, 512)],
                                              jnp.max(v))

    @pl.when(g == 15)
    def _():
        col = lax.broadcasted_iota(jnp.int32, (D, _TCB), 1)
        valid = (15 * _TCB + col) < N
        vmin = jnp.min(jnp.where(valid, v, jnp.inf))
        vmax = jnp.max(jnp.where(valid, v, -jnp.inf))
        mm_ref[pl.ds(0, 512)] = jnp.minimum(mm_ref[pl.ds(0, 512)], vmin)
        mm_ref[pl.ds(512, 512)] = jnp.maximum(mm_ref[pl.ds(512, 512)], vmax)


@functools.partial(
    pl.kernel,
    mesh=_mesh,
    compiler_params=pltpu.CompilerParams(needs_layout_passes=False),
    out_type=jax.ShapeDtypeStruct((NW * NB,), jnp.float32),
    scratch_types=[
        pltpu.VMEM((8, CC), jnp.float32),
        pltpu.VMEM((8, CC), jnp.float32),
        pltpu.VMEM((NB,), jnp.float32),
        pltpu.VMEM((NB,), jnp.float32),
        pltpu.VMEM((NB,), jnp.float32),
        pltpu.VMEM((NB,), jnp.float32),
        pltpu.VMEM((NW * 32,), jnp.float32),
        pltpu.SemaphoreType.DMA,
        pltpu.SemaphoreType.DMA,
    ],
)
def _hist_k(x_hbm, xt_hbm, mm_hbm, hist_hbm, buf0, buf1, h0, h1, h2, h3,
            mm_v, sem0, sem1):
    w = _wid()
    c16, r0, base_c = _worker_slices(w)
    pltpu.async_copy(
        x_hbm.at[pl.ds(r0, 8), pl.ds(base_c, CC)], buf0, sem0)
    pltpu.sync_copy(mm_hbm, mm_v)
    lo_v, hi_v = _reduce_minmax(mm_v)
    width_v = (hi_v - lo_v) * (1.0 / NB)
    wpe_v = width_v + EPS

    hists = (h0, h1, h2, h3)
    zero = jnp.zeros((16,), jnp.float32)
    for i in range(NB // 16):
        for h in hists:
            h[pl.ds(16 * i, 16)] = zero
    ones = jnp.full((16,), 1.0, jnp.float32)
    topf = jnp.full((16,), float(NB - 1), jnp.float32)

    def chunk_pair(g, carry):
        for b in range(2):
            c = 2 * g + b
            buf = buf0 if b == 0 else buf1
            sem = sem0 if b == 0 else sem1
            nbuf = buf1 if b == 0 else buf0
            nsem = sem1 if b == 0 else sem0

            @pl.when(c + 1 < NCH)
            def _():
                col = pl.multiple_of(base_c + (c + 1) * CC, 128)
                pltpu.async_copy(
                    x_hbm.at[pl.ds(r0, 8), pl.ds(col, CC)], nbuf, nsem)

            pltpu.make_async_copy(
                x_hbm.at[pl.ds(0, 8), pl.ds(0, CC)], buf, sem).wait()

            for r in range(8):
                @plsc.parallel_loop(0, INNER, 1, unroll=2)
                def body(i, _r=r, _buf=buf):
                    for u in range(UNROLL):
                        v = _buf[_r, pl.ds((i * UNROLL + u) * 16, 16)]
                        q = (v - lo_v) / wpe_v
                        idx = jnp.minimum(q, topf).astype(jnp.int32)
                        plsc.addupdate_scatter(hists[u], [idx], ones)
        return carry

    lax.fori_loop(0, NCH // 2, chunk_pair, 0)

    @pl.when(w == 0)
    def _():
        pltpu.sync_copy(xt_hbm.at[pl.ds(0, TA)], buf0.at[0, pl.ds(0, TA)])
        pltpu.sync_copy(xt_hbm.at[pl.ds(TA, TB)], buf0.at[1, pl.ds(0, TB)])
        for r, nvr in ((0, TA // 64), (1, TB // 64)):
            @plsc.parallel_loop(0, nvr, 1)
            def body(i, _r=r):
                for u in range(UNROLL):
                    v = buf0[_r, pl.ds((i * UNROLL + u) * 16, 16)]
                    q = (v - lo_v) / wpe_v
                    idx = jnp.minimum(q, topf).astype(jnp.int32)
                    plsc.addupdate_scatter(hists[u], [idx], ones)

    def merge(i, carry):
        s = (h0[pl.ds(16 * i, 16)] + h1[pl.ds(16 * i, 16)]
             + h2[pl.ds(16 * i, 16)] + h3[pl.ds(16 * i, 16)])
        h0[pl.ds(16 * i, 16)] = s
        return carry

    lax.fori_loop(0, NB // 16, merge, 0)
    pltpu.sync_copy(h0, hist_hbm.at[pl.ds(NB * w, NB)])


@functools.partial(
    pl.kernel,
    mesh=_mesh,
    compiler_params=pltpu.CompilerParams(needs_layout_passes=False),
    out_type=jax.ShapeDtypeStruct((1040,), jnp.float32),
    scratch_types=[
        pltpu.VMEM((NW * NB,), jnp.float32),
        pltpu.VMEM((NB,), jnp.float32),
        pltpu.VMEM((NB,), jnp.float32),
        pltpu.VMEM((1040,), jnp.float32),
        pltpu.VMEM((NW * 32,), jnp.float32),
    ],
)
def _final_k(hist_hbm, mm_hbm, q_hbm, hall, acc, cdfm, qout, mm_v):
    w = _wid()

    @pl.when(w == 0)
    def _():
        pltpu.sync_copy(hist_hbm, hall)
        pltpu.sync_copy(mm_hbm, mm_v)
        lo_v, hi_v = _reduce_minmax(mm_v)
        width_v = (hi_v - lo_v) * (1.0 / NB)

        @plsc.parallel_loop(0, NB // 16, 1, unroll=2)
        def merge(i):
            s = jnp.zeros((16,), jnp.float32)
            for j in range(NW):
                s = s + hall[pl.ds(j * NB + i * 16, 16)]
            acc[pl.ds(16 * i, 16)] = s

        def csum(i, carry):
            v = acc[pl.ds(16 * i, 16)]
            cs = plsc.cumsum(v) + carry
            cdfm[pl.ds(16 * i, 16)] = cs
            return _bcast_last_max(cs)

        lax.fori_loop(0, NB // 16, csum, jnp.zeros((16,), jnp.float32))

        lane = lax.iota(jnp.int32, 16)
        zero_i = jnp.zeros((16,), jnp.int32)
        zero_f = jnp.zeros((16,), jnp.float32)

        @plsc.parallel_loop(0, 1040 // 16, 1, unroll=2)
        def interp(i):
            k = lane + 16 * i
            tq = k.astype(jnp.float32) * (TOT / NB)
            lo_i = zero_i
            hi_i = jnp.full((16,), NB + 1, jnp.int32)
            for _ in range(11):
                mid = (lo_i + hi_i) >> 1
                cm = plsc.load_gather(cdfm, [jnp.maximum(mid - 1, zero_i)])
                cval = jnp.where(mid == 0, zero_f, cm)
                pred = cval < tq
                lo_i = jnp.where(pred, mid + 1, lo_i)
                hi_i = jnp.where(pred, hi_i, mid)
            ind = jnp.clip(lo_i - 1, 0, NB - 1)
            indf = ind.astype(jnp.float32)
            e1 = lo_v + indf * width_v
            e2 = lo_v + (indf + 1.0) * width_v
            cs_lo = plsc.load_gather(cdfm, [jnp.maximum(ind - 1, zero_i)])
            cs_lo = jnp.where(ind == 0, zero_f, cs_lo)
            cs_hi = plsc.load_gather(cdfm, [ind])
            t = k.astype(jnp.float32) * (1.0 / NB)
            slope = (e2 - e1) / (EPS + (cs_hi * (1.0 / TOT) - cs_lo * (1.0 / TOT)))
            qv = e1 + slope * (t - cs_lo * (1.0 / TOT))
            qout[pl.ds(16 * i, 16)] = qv

        pltpu.sync_copy(qout, q_hbm)


def kernel(x):
    xt = x[:, TAIL0:].reshape(-1)
    mm = _minmax_tc(x)
    hists = _hist_k(x, xt, mm)
    q = _final_k(hists, mm)
    return q[:1025]
